# Initial kernel scaffold; baseline (speedup 1.0000x reference)
import functools

import jax
import jax.numpy as jnp
from jax import lax
from jax.experimental import pallas as pl
from jax.experimental.pallas import tpu as pltpu
from jax.experimental.pallas import tpu_sc as plsc

FM = 128
NL = 50000
NC = 50000
NV = 25000
NE = 600000
NE_PAD = 600064
EPT = NE_PAD // 16
EBLK = EPT // 8
CHUNK = 12800
NCHUNK = 4
OUT_ROWS = CHUNK * NCHUNK
ACC_ROWS = CHUNK + 16
WL = 16512
BATCH = 128
DUMP = CHUNK
INV_SQRT = float(1.0 / (128.0 ** 0.5))
LN_EPS = 1e-6
EPS_LOSS = 1e-8

_f32 = jnp.float32
_i32 = jnp.int32

_MESH = plsc.VectorSubcoreMesh(
    core_axis_name="c", subcore_axis_name="s", num_cores=2, num_subcores=16
)


def _sc_attention_body(r_hbm, s_hbm, qp_hbm, x_hbm, acc_hbm, den_hbm,
                       acc_sp, den_sp, ebuf_r, ebuf_s, wl_r, wl_s,
                       ridx, sidx, gidx, xbuf, qbuf, sbuf, wbuf):
    tile = lax.axis_index("s")
    core = lax.axis_index("c")
    zero16 = jnp.zeros((16,), _f32)

    def chunk_body(i, _):
        ch = core + 2 * i
        lo = ch * CHUNK
        hi = jnp.minimum(lo + CHUNK, NL)

        def zrow(r, _):
            for j in range(8):
                sbuf[r, pl.ds(j * 16, 16)] = zero16
            return 0
        lax.fori_loop(0, BATCH, zrow, 0)

        base = tile * 800
        for p in range(6):
            pltpu.sync_copy(sbuf, acc_sp.at[pl.ds(base + p * 128, 128)])
        pltpu.sync_copy(sbuf.at[pl.ds(0, 32)], acc_sp.at[pl.ds(base + 768, 32)])
        for p in range(6):
            pltpu.sync_copy(sbuf.at[0], den_sp.at[pl.ds(base + p * 128, 128)])
        pltpu.sync_copy(sbuf.at[0, pl.ds(0, 32)], den_sp.at[pl.ds(base + 768, 32)])
        plsc.subcore_barrier()

        def eblk_body(blk, cnt):
            ebase = tile * EPT + blk * EBLK
            pltpu.sync_copy(r_hbm.at[pl.ds(ebase, EBLK)], ebuf_r)
            pltpu.sync_copy(s_hbm.at[pl.ds(ebase, EBLK)], ebuf_s)

            def grp(gi, cnt):
                rv = ebuf_r[pl.ds(gi * 16, 16)]
                sv = ebuf_s[pl.ds(gi * 16, 16)]
                m = (rv >= lo) & (rv < hi)
                plsc.store_compressed(wl_r.at[pl.ds(cnt, 16)], rv - lo, m)
                plsc.store_compressed(wl_s.at[pl.ds(cnt, 16)], sv, m)
                return cnt + jnp.sum(m.astype(_i32))
            return lax.fori_loop(0, EBLK // 16, grp, cnt)
        cnt = lax.fori_loop(0, 8, eblk_body, jnp.int32(0))

        for j in range(8):
            wl_r[pl.ds(cnt + j * 16, 16)] = jnp.full((16,), DUMP, _i32)
            wl_s[pl.ds(cnt + j * 16, 16)] = jnp.zeros((16,), _i32)
        nb = (cnt + BATCH - 1) // BATCH

        def batch(j, _):
            off = j * BATCH
            for gi in range(8):
                rl = wl_r[pl.ds(off + gi * 16, 16)]
                ridx[pl.ds(gi * 16, 16)] = rl
                gidx[pl.ds(gi * 16, 16)] = jnp.minimum(rl + lo, NL - 1)
                sidx[pl.ds(gi * 16, 16)] = wl_s[pl.ds(off + gi * 16, 16)]
            pltpu.sync_copy(x_hbm.at[sidx], xbuf)
            pltpu.sync_copy(qp_hbm.at[gidx], qbuf)

            def grp_dot(gi, _):
                eids = lax.iota(_i32, 16) + gi * 16
                acc = jnp.zeros((16,), _f32)
                for d in range(FM):
                    dcol = jnp.full((16,), d, _i32)
                    xc = plsc.load_gather(xbuf, [eids, dcol])
                    qc = plsc.load_gather(qbuf, [eids, dcol])
                    acc = acc + xc * qc
                wbuf[pl.ds(gi * 16, 16)] = jnp.exp(acc * INV_SQRT)
                return 0
            lax.fori_loop(0, 8, grp_dot, 0)

            def scale(e, _):
                w = wbuf[e]
                for d in range(8):
                    sbuf[e, pl.ds(d * 16, 16)] = xbuf[e, pl.ds(d * 16, 16)] * w
                return 0
            lax.fori_loop(0, BATCH, scale, 0)

            pltpu.sync_copy(sbuf, acc_sp.at[ridx], add=True)
            pltpu.sync_copy(wbuf, den_sp.at[ridx], add=True)
            return 0
        lax.fori_loop(0, nb, batch, 0)
        plsc.subcore_barrier()

        obase = lo + tile * 800
        for p in range(6):
            pltpu.sync_copy(acc_sp.at[pl.ds(base + p * 128, 128)],
                            acc_hbm.at[pl.ds(obase + p * 128, 128)])
        pltpu.sync_copy(acc_sp.at[pl.ds(base + 768, 32)],
                        acc_hbm.at[pl.ds(obase + 768, 32)])
        for p in range(6):
            pltpu.sync_copy(den_sp.at[pl.ds(base + p * 128, 128)],
                            den_hbm.at[pl.ds(obase + p * 128, 128)])
        pltpu.sync_copy(den_sp.at[pl.ds(base + 768, 32)],
                        den_hbm.at[pl.ds(obase + 768, 32)])
        return 0

    lax.fori_loop(0, 2, chunk_body, 0)


_sc_attention = pl.kernel(
    _sc_attention_body,
    out_type=(
        jax.ShapeDtypeStruct((OUT_ROWS, FM), _f32),
        jax.ShapeDtypeStruct((OUT_ROWS,), _f32),
    ),
    mesh=_MESH,
    scratch_types=[
        pltpu.VMEM_SHARED((ACC_ROWS, FM), _f32),
        pltpu.VMEM_SHARED((ACC_ROWS,), _f32),
        pltpu.VMEM((EBLK,), _i32),
        pltpu.VMEM((EBLK,), _i32),
        pltpu.VMEM((WL,), _i32),
        pltpu.VMEM((WL,), _i32),
        pltpu.VMEM((BATCH,), _i32),
        pltpu.VMEM((BATCH,), _i32),
        pltpu.VMEM((BATCH,), _i32),
        pltpu.VMEM((BATCH, FM), _f32),
        pltpu.VMEM((BATCH, FM), _f32),
        pltpu.VMEM((BATCH, FM), _f32),
        pltpu.VMEM((BATCH,), _f32),
    ],
)


def _sc_segsum_body(r_hbm, s_hbm, val_hbm, out_hbm,
                    sum_sp, ebuf_r, ebuf_s, wl_r, wl_s, ridx, sidx, vbuf, zbuf):
    tile = lax.axis_index("s")
    core = lax.axis_index("c")
    zero16 = jnp.zeros((16,), _f32)

    def chunk_body(i, _):
        ch = core + 2 * i
        lo = ch * CHUNK
        hi = jnp.minimum(lo + CHUNK, NC)

        for j in range(8):
            zbuf[pl.ds(j * 16, 16)] = zero16
        base = tile * 800
        for p in range(6):
            pltpu.sync_copy(zbuf, sum_sp.at[pl.ds(base + p * 128, 128)])
        pltpu.sync_copy(zbuf.at[pl.ds(0, 32)], sum_sp.at[pl.ds(base + 768, 32)])
        plsc.subcore_barrier()

        def eblk_body(blk, cnt):
            ebase = tile * EPT + blk * EBLK
            pltpu.sync_copy(r_hbm.at[pl.ds(ebase, EBLK)], ebuf_r)
            pltpu.sync_copy(s_hbm.at[pl.ds(ebase, EBLK)], ebuf_s)

            def grp(gi, cnt):
                rv = ebuf_r[pl.ds(gi * 16, 16)]
                sv = ebuf_s[pl.ds(gi * 16, 16)]
                m = (rv >= lo) & (rv < hi)
                plsc.store_compressed(wl_r.at[pl.ds(cnt, 16)], rv - lo, m)
                plsc.store_compressed(wl_s.at[pl.ds(cnt, 16)], sv, m)
                return cnt + jnp.sum(m.astype(_i32))
            return lax.fori_loop(0, EBLK // 16, grp, cnt)
        cnt = lax.fori_loop(0, 8, eblk_body, jnp.int32(0))

        for j in range(8):
            wl_r[pl.ds(cnt + j * 16, 16)] = jnp.full((16,), DUMP, _i32)
            wl_s[pl.ds(cnt + j * 16, 16)] = jnp.zeros((16,), _i32)
        nb = (cnt + BATCH - 1) // BATCH

        def batch(j, _):
            off = j * BATCH
            for gi in range(8):
                ridx[pl.ds(gi * 16, 16)] = wl_r[pl.ds(off + gi * 16, 16)]
                sidx[pl.ds(gi * 16, 16)] = wl_s[pl.ds(off + gi * 16, 16)]
            pltpu.sync_copy(val_hbm.at[sidx], vbuf)
            pltpu.sync_copy(vbuf, sum_sp.at[ridx], add=True)
            return 0
        lax.fori_loop(0, nb, batch, 0)
        plsc.subcore_barrier()

        obase = lo + tile * 800
        for p in range(6):
            pltpu.sync_copy(sum_sp.at[pl.ds(base + p * 128, 128)],
                            out_hbm.at[pl.ds(obase + p * 128, 128)])
        pltpu.sync_copy(sum_sp.at[pl.ds(base + 768, 32)],
                        out_hbm.at[pl.ds(obase + 768, 32)])
        return 0

    lax.fori_loop(0, 2, chunk_body, 0)


_sc_segsum = pl.kernel(
    _sc_segsum_body,
    out_type=jax.ShapeDtypeStruct((OUT_ROWS,), _f32),
    mesh=_MESH,
    scratch_types=[
        pltpu.VMEM_SHARED((ACC_ROWS,), _f32),
        pltpu.VMEM((EBLK,), _i32),
        pltpu.VMEM((EBLK,), _i32),
        pltpu.VMEM((WL,), _i32),
        pltpu.VMEM((WL,), _i32),
        pltpu.VMEM((BATCH,), _i32),
        pltpu.VMEM((BATCH,), _i32),
        pltpu.VMEM((BATCH,), _f32),
        pltpu.VMEM((BATCH,), _f32),
    ],
)


def _ln(x, g, b):
    m = jnp.mean(x, axis=-1, keepdims=True)
    v = jnp.mean((x - m) ** 2, axis=-1, keepdims=True)
    return (x - m) * lax.rsqrt(v + LN_EPS) * g + b


def _softplus(x):
    return jnp.maximum(x, 0.0) + jnp.log1p(jnp.exp(-jnp.abs(x)))


def _qprime_body(emb_ref, w0_ref, w1_ref, out_ref):
    a = lax.dot_general(w0_ref[...], w1_ref[...], (((1,), (1,)), ((), ())),
                        preferred_element_type=_f32)
    out_ref[...] = jnp.dot(emb_ref[...], a, preferred_element_type=_f32)


def _qprime(emb, w0, w1):
    return pl.pallas_call(
        _qprime_body,
        grid=(25,),
        in_specs=[pl.BlockSpec((2000, FM), lambda i: (i, 0)),
                  pl.BlockSpec((FM, FM), lambda i: (0, 0)),
                  pl.BlockSpec((FM, FM), lambda i: (0, 0))],
        out_specs=pl.BlockSpec((2000, FM), lambda i: (i, 0)),
        out_shape=jax.ShapeDtypeStruct((NL, FM), _f32),
    )(emb, w0, w1)


def _post_body(old_ref, acc_ref, den_ref, w2_ref, g0_ref, b0_ref,
               m0_ref, m1_ref, m2_ref, mb0_ref, mb1_ref, mb2_ref,
               g1_ref, b1_ref, out_ref):
    new = jnp.dot(acc_ref[...], w2_ref[...], preferred_element_type=_f32)
    new = new / (den_ref[...] + 1e-9)
    t = _ln(old_ref[...] + new, g0_ref[...], b0_ref[...])
    h = jax.nn.relu(jnp.dot(t, m0_ref[...], preferred_element_type=_f32) + mb0_ref[...])
    h = jax.nn.relu(jnp.dot(h, m1_ref[...], preferred_element_type=_f32) + mb1_ref[...])
    h = jnp.dot(h, m2_ref[...], preferred_element_type=_f32) + mb2_ref[...]
    out_ref[...] = _ln(t + h, g1_ref[...], b1_ref[...])


def _mat_spec(i):
    return pl.BlockSpec((FM, FM), lambda i: (0, 0))


_W_SPECS = [pl.BlockSpec((FM, FM), lambda i: (0, 0))] * 4 + \
           [pl.BlockSpec((1, FM), lambda i: (0, 0))] * 2


def _post_specs(br, old_map, acc_map):
    full_m = lambda i: (0, 0)
    return [
        pl.BlockSpec((br, FM), old_map),
        pl.BlockSpec((br, FM), acc_map),
        pl.BlockSpec((br, 1), acc_map),
        pl.BlockSpec((FM, FM), full_m),
        pl.BlockSpec((1, FM), full_m),
        pl.BlockSpec((1, FM), full_m),
        pl.BlockSpec((FM, FM), full_m),
        pl.BlockSpec((FM, FM), full_m),
        pl.BlockSpec((FM, FM), full_m),
        pl.BlockSpec((1, FM), full_m),
        pl.BlockSpec((1, FM), full_m),
        pl.BlockSpec((1, FM), full_m),
        pl.BlockSpec((1, FM), full_m),
        pl.BlockSpec((1, FM), full_m),
    ]


def _post_c(old, acc, den, *weights):
    ident = lambda i: (i, 0)
    return pl.pallas_call(
        _post_body,
        grid=(25,),
        in_specs=_post_specs(2000, ident, ident),
        out_specs=pl.BlockSpec((2000, FM), ident),
        out_shape=jax.ShapeDtypeStruct((NC, FM), _f32),
    )(old, acc, den, *weights)


def _post_l(old, acc, den, *weights):
    ident = lambda i: (i, 0)
    flip = lambda i: (jnp.where(i < 25, i + 25, i - 25), 0)
    return pl.pallas_call(
        _post_body,
        grid=(50,),
        in_specs=_post_specs(1000, ident, flip),
        out_specs=pl.BlockSpec((1000, FM), ident),
        out_shape=jax.ShapeDtypeStruct((NL, FM), _f32),
    )(old, acc, den, *weights)


def _readout_body(lp_ref, ln_ref, w1_ref, b1_ref, w2_ref, b2_ref,
                  w3_ref, b3_ref, log_ref, sp_ref, sn_ref):
    v = jnp.concatenate([lp_ref[...], ln_ref[...]], axis=1)
    h = jax.nn.relu(jnp.dot(v, w1_ref[...], preferred_element_type=_f32) + b1_ref[...])
    h = jax.nn.relu(jnp.dot(h, w2_ref[...], preferred_element_type=_f32) + b2_ref[...])
    lg = jnp.dot(h, w3_ref[...], preferred_element_type=_f32) + b3_ref[...]
    log_ref[...] = lg
    sp_ref[...] = _softplus(lg)
    sn_ref[...] = _softplus(-lg)


def _readout(l_out, w1, b1, w2, b2, w3, b3):
    full_m = lambda i: (0, 0)
    return pl.pallas_call(
        _readout_body,
        grid=(25,),
        in_specs=[
            pl.BlockSpec((1000, FM), lambda i: (i, 0)),
            pl.BlockSpec((1000, FM), lambda i: (i + 25, 0)),
            pl.BlockSpec((2 * FM, 2 * FM), full_m),
            pl.BlockSpec((1, 2 * FM), full_m),
            pl.BlockSpec((2 * FM, 2 * FM), full_m),
            pl.BlockSpec((1, 2 * FM), full_m),
            pl.BlockSpec((2 * FM, 1), full_m),
            pl.BlockSpec((1, 1), full_m),
        ],
        out_specs=[
            pl.BlockSpec((1000, 1), lambda i: (i, 0)),
            pl.BlockSpec((1000, 1), lambda i: (i, 0)),
            pl.BlockSpec((1000, 1), lambda i: (i, 0)),
        ],
        out_shape=[
            jax.ShapeDtypeStruct((NV, 1), _f32),
            jax.ShapeDtypeStruct((NV, 1), _f32),
            jax.ShapeDtypeStruct((NV, 1), _f32),
        ],
    )(l_out, l_out, w1, b1, w2, b2, w3, b3)


def _loss_body(s_ref, out_ref):
    s = s_ref[...]
    cv = jnp.exp(-s)
    lg = jnp.log(1.0 - cv + EPS_LOSS)
    out_ref[...] = jnp.sum(lg * lg).reshape(1, 1)


def _loss(s_2d):
    return pl.pallas_call(
        _loss_body,
        out_shape=jax.ShapeDtypeStruct((1, 1), _f32),
    )(s_2d)


def kernel(edge_index, n_lits, n_clauses, L_init, C_init, lit_mlp_W, lit_mlp_b,
           cl_mlp_W, cl_mlp_b, att_l_W, att_c_W, ln_gamma, ln_beta,
           out_W1, out_b1, out_W2, out_b2, out_W3, out_b3):
    lit = edge_index[0]
    cl = edge_index[1]
    npad = NE_PAD - NE
    big = jnp.full((npad,), 2 ** 30, _i32)
    zpad = jnp.zeros((npad,), _i32)
    lit_r = jnp.concatenate([lit, big])
    lit_s = jnp.concatenate([lit, zpad])
    cl_r = jnp.concatenate([cl, big])
    cl_s = jnp.concatenate([cl, zpad])

    denom = jnp.sqrt(jnp.float32(FM))
    l_out = jnp.tile((L_init / denom).astype(_f32), (NL, 1))
    c_out = jnp.tile((C_init / denom).astype(_f32), (NC, 1))

    g = [ln_gamma[i].reshape(1, FM) for i in range(4)]
    b = [ln_beta[i].reshape(1, FM) for i in range(4)]
    cw = [cl_mlp_W[i] for i in range(3)]
    cb = [cl_mlp_b[i].reshape(1, FM) for i in range(3)]
    lw = [lit_mlp_W[i] for i in range(3)]
    lb = [lit_mlp_b[i].reshape(1, FM) for i in range(3)]
    b1r = out_b1.reshape(1, 2 * FM)
    b2r = out_b2.reshape(1, 2 * FM)
    b3r = out_b3.reshape(1, 1)

    losses = []
    logits = None
    for _ in range(2):
        qc = _qprime(c_out, att_c_W[0], att_c_W[1])
        accc, denc = _sc_attention(cl_r, lit_s, qc, l_out)
        c_out = _post_c(c_out, accc, denc.reshape(OUT_ROWS, 1), att_c_W[2],
                        g[0], b[0], cw[0], cw[1], cw[2], cb[0], cb[1], cb[2],
                        g[3], b[3])
        ql = _qprime(l_out, att_l_W[0], att_l_W[1])
        accl, denl = _sc_attention(lit_r, cl_s, ql, c_out)
        l_out = _post_l(l_out, accl, denl.reshape(OUT_ROWS, 1), att_l_W[2],
                        g[1], b[1], lw[0], lw[1], lw[2], lb[0], lb[1], lb[2],
                        g[2], b[2])
        logits, sp_pos, sp_neg = _readout(l_out, out_W1, b1r, out_W2, b2r,
                                          out_W3, b3r)
        lit_sp = jnp.concatenate([sp_pos, sp_neg], axis=0).reshape(NL)
        s_c = _sc_segsum(cl_r, lit_s, lit_sp)
        losses.append(_loss(s_c[:NC].reshape(400, 125))[0, 0])
    return logits, jnp.mean(jnp.stack(losses))


# speedup vs baseline: 15.6201x; 15.6201x over previous
import functools

import jax
import jax.numpy as jnp
import numpy as _np
from jax import lax
from jax.experimental import pallas as pl
from jax.experimental.pallas import tpu as pltpu
from jax.experimental.pallas import tpu_sc as plsc

FM = 128
NL = 50000
NC = 50000
NV = 25000
NE = 600000
NE_PAD = 600064
EPT = NE_PAD // 16
EBLK = EPT // 8
CHUNK = 10240
NCHUNK = 6
OUT_ROWS = CHUNK * NCHUNK
ACC_ROWS = CHUNK + 16
WL = 10240
BATCH = 64
GRPS = BATCH // 16
DUMP = CHUNK
INV_SQRT = float(1.0 / (128.0 ** 0.5))
LN_EPS = 1e-6
EPS_LOSS = 1e-8

_f32 = jnp.float32
_i32 = jnp.int32

_MESH = plsc.VectorSubcoreMesh(
    core_axis_name="c", subcore_axis_name="s", num_cores=2, num_subcores=16
)


def _sc_attention_body(r_hbm, s_hbm, qp_hbm, x_hbm, acc_hbm, den_hbm,
                       acc_sp, den_sp, ebuf_r, ebuf_s, wl, ridx, idxbuf,
                       xbuf, qbuf, wbuf):
    tile = lax.axis_index("s")
    core = lax.axis_index("c")
    zero16 = jnp.zeros((16,), _f32)
    lane = lax.iota(_i32, 16)
    perms = [(lane + sh) % 16 for sh in (8, 4, 2, 1)]

    def chunk_body(i, _):
        ch = core + 2 * i
        lo = ch * CHUNK
        hi = jnp.minimum(lo + CHUNK, NL)

        def zrow(r, _):
            for j in range(8):
                xbuf[r, pl.ds(j * 16, 16)] = zero16
            return 0
        lax.fori_loop(0, BATCH, zrow, 0)

        base = tile * 640
        for p in range(10):
            pltpu.sync_copy(xbuf, acc_sp.at[pl.ds(base + p * 64, 64)])
        for p in range(5):
            pltpu.sync_copy(xbuf.at[0], den_sp.at[pl.ds(base + p * 128, 128)])
        plsc.subcore_barrier()

        def eblk_body(blk, cnt):
            ebase = tile * EPT + blk * EBLK
            pltpu.sync_copy(r_hbm.at[pl.ds(ebase, EBLK)], ebuf_r)
            pltpu.sync_copy(s_hbm.at[pl.ds(ebase, EBLK)], ebuf_s)

            def grp(gi, cnt):
                rv = ebuf_r[pl.ds(gi * 16, 16)]
                sv = ebuf_s[pl.ds(gi * 16, 16)]
                m = (rv >= lo) & (rv < hi)
                mi = m.astype(_i32)
                pos = cnt + plsc.cumsum(mi) - 1
                plsc.store_scatter(wl, [pos], (rv - lo) * 65536 + sv, mask=m)
                return cnt + jnp.sum(mi)
            return lax.fori_loop(0, EBLK // 16, grp, cnt)
        cnt = lax.fori_loop(0, 8, eblk_body, jnp.int32(0))

        for j in range(GRPS):
            wl[pl.ds(cnt + j * 16, 16)] = jnp.full((16,), DUMP * 65536, _i32)
        nb = (cnt + BATCH - 1) // BATCH

        def batch(j, _):
            off = j * BATCH
            for gi in range(GRPS):
                wp = wl[pl.ds(off + gi * 16, 16)]
                rl = wp >> 16
                sv = wp & 65535
                ridx[pl.ds(gi * 16, 16)] = rl
                idxbuf[pl.ds(gi * 16, 16)] = jnp.minimum(rl + lo, NL - 1)
                idxbuf[pl.ds(BATCH + gi * 16, 16)] = sv
            pltpu.sync_copy(x_hbm.at[idxbuf.at[pl.ds(BATCH, BATCH)]], xbuf)
            pltpu.sync_copy(qp_hbm.at[idxbuf.at[pl.ds(0, BATCH)]], qbuf)

            def grp_dot(gi, _):
                wg = zero16
                for l16 in range(16):
                    e = gi * 16 + l16
                    xs = [xbuf[e, pl.ds(d * 16, 16)] for d in range(8)]
                    p = xs[0] * qbuf[e, pl.ds(0, 16)]
                    for d in range(1, 8):
                        p = p + xs[d] * qbuf[e, pl.ds(d * 16, 16)]
                    for pm in perms:
                        p = p + jnp.take(p, pm)
                    wv = jnp.exp(p * INV_SQRT)
                    for d in range(8):
                        xbuf[e, pl.ds(d * 16, 16)] = xs[d] * wv
                    wg = wg + wv * (lane == l16).astype(_f32)
                wbuf[pl.ds(gi * 16, 16)] = wg
                return 0
            lax.fori_loop(0, GRPS, grp_dot, 0)

            pltpu.sync_copy(xbuf, acc_sp.at[ridx], add=True)
            pltpu.sync_copy(wbuf, den_sp.at[ridx], add=True)
            return 0
        lax.fori_loop(0, nb, batch, 0)
        plsc.subcore_barrier()

        obase = lo + tile * 640
        for p in range(5):
            pltpu.sync_copy(acc_sp.at[pl.ds(base + p * 128, 128)],
                            acc_hbm.at[pl.ds(obase + p * 128, 128)])
            pltpu.sync_copy(den_sp.at[pl.ds(base + p * 128, 128)],
                            den_hbm.at[pl.ds(obase + p * 128, 128)])
        plsc.subcore_barrier()
        return 0

    lax.fori_loop(0, 3, chunk_body, 0)


_sc_attention = pl.kernel(
    _sc_attention_body,
    out_type=(
        jax.ShapeDtypeStruct((OUT_ROWS, FM), _f32),
        jax.ShapeDtypeStruct((OUT_ROWS,), _f32),
    ),
    mesh=_MESH,
    compiler_params=pltpu.CompilerParams(needs_layout_passes=False),
    scratch_types=[
        pltpu.VMEM_SHARED((ACC_ROWS, FM), _f32),
        pltpu.VMEM_SHARED((ACC_ROWS,), _f32),
        pltpu.VMEM((EBLK,), _i32),
        pltpu.VMEM((EBLK,), _i32),
        pltpu.VMEM((WL,), _i32),
        pltpu.VMEM((BATCH,), _i32),
        pltpu.VMEM((2 * BATCH,), _i32),
        pltpu.VMEM((BATCH, FM), _f32),
        pltpu.VMEM((BATCH, FM), _f32),
        pltpu.VMEM((BATCH,), _f32),
    ],
)


def _sc_segsum_body(r_hbm, s_hbm, val_hbm, out_hbm,
                    sum_sp, ebuf_r, ebuf_s, wl, ridx, sidx, vbuf, zbuf):
    tile = lax.axis_index("s")
    core = lax.axis_index("c")
    zero16 = jnp.zeros((16,), _f32)

    def chunk_body(i, _):
        ch = core + 2 * i
        lo = ch * CHUNK
        hi = jnp.minimum(lo + CHUNK, NC)

        for j in range(8):
            zbuf[pl.ds(j * 16, 16)] = zero16
        for k in range(5):
            pltpu.sync_copy(zbuf, sum_sp.at[pl.ds((tile * 5 + k) * 128, 128)])
        plsc.subcore_barrier()

        def eblk_body(blk, cnt):
            ebase = tile * EPT + blk * EBLK
            pltpu.sync_copy(r_hbm.at[pl.ds(ebase, EBLK)], ebuf_r)
            pltpu.sync_copy(s_hbm.at[pl.ds(ebase, EBLK)], ebuf_s)

            def grp(gi, cnt):
                rv = ebuf_r[pl.ds(gi * 16, 16)]
                sv = ebuf_s[pl.ds(gi * 16, 16)]
                m = (rv >= lo) & (rv < hi)
                mi = m.astype(_i32)
                pos = cnt + plsc.cumsum(mi) - 1
                plsc.store_scatter(wl, [pos], (rv - lo) * 65536 + sv, mask=m)
                return cnt + jnp.sum(mi)
            return lax.fori_loop(0, EBLK // 16, grp, cnt)
        cnt = lax.fori_loop(0, 8, eblk_body, jnp.int32(0))

        for j in range(GRPS):
            wl[pl.ds(cnt + j * 16, 16)] = jnp.full((16,), DUMP * 65536, _i32)
        nb = (cnt + BATCH - 1) // BATCH

        def batch(j, _):
            off = j * BATCH
            for gi in range(GRPS):
                wp = wl[pl.ds(off + gi * 16, 16)]
                ridx[pl.ds(gi * 16, 16)] = wp >> 16
                sidx[pl.ds(gi * 16, 16)] = wp & 65535
            pltpu.sync_copy(val_hbm.at[sidx], vbuf)
            pltpu.sync_copy(vbuf, sum_sp.at[ridx], add=True)
            return 0
        lax.fori_loop(0, nb, batch, 0)
        plsc.subcore_barrier()

        for k in range(5):
            pltpu.sync_copy(sum_sp.at[pl.ds((tile * 5 + k) * 128, 128)],
                            out_hbm.at[pl.ds(lo + (tile * 5 + k) * 128, 128)])
        plsc.subcore_barrier()
        return 0

    lax.fori_loop(0, 3, chunk_body, 0)


_sc_segsum = pl.kernel(
    _sc_segsum_body,
    out_type=jax.ShapeDtypeStruct((OUT_ROWS,), _f32),
    mesh=_MESH,
    compiler_params=pltpu.CompilerParams(needs_layout_passes=False),
    scratch_types=[
        pltpu.VMEM_SHARED((ACC_ROWS,), _f32),
        pltpu.VMEM((EBLK,), _i32),
        pltpu.VMEM((EBLK,), _i32),
        pltpu.VMEM((WL,), _i32),
        pltpu.VMEM((BATCH,), _i32),
        pltpu.VMEM((BATCH,), _i32),
        pltpu.VMEM((BATCH,), _f32),
        pltpu.VMEM((128,), _f32),
    ],
)


def _ln(x, g, b):
    m = jnp.mean(x, axis=-1, keepdims=True)
    v = jnp.mean((x - m) ** 2, axis=-1, keepdims=True)
    return (x - m) * lax.rsqrt(v + LN_EPS) * g + b


def _softplus(x):
    return jnp.maximum(x, 0.0) + jnp.log1p(jnp.exp(-jnp.abs(x)))


def _qprime_body(emb_ref, w0_ref, w1_ref, out_ref):
    a = lax.dot_general(w0_ref[...], w1_ref[...], (((1,), (1,)), ((), ())),
                        preferred_element_type=_f32)
    out_ref[...] = jnp.dot(emb_ref[...], a, preferred_element_type=_f32)


def _qprime(emb, w0, w1):
    return pl.pallas_call(
        _qprime_body,
        grid=(25,),
        in_specs=[pl.BlockSpec((2000, FM), lambda i: (i, 0)),
                  pl.BlockSpec((FM, FM), lambda i: (0, 0)),
                  pl.BlockSpec((FM, FM), lambda i: (0, 0))],
        out_specs=pl.BlockSpec((2000, FM), lambda i: (i, 0)),
        out_shape=jax.ShapeDtypeStruct((NL, FM), _f32),
    )(emb, w0, w1)


def _post_body(old_ref, acc_ref, den_ref, w2_ref, g0_ref, b0_ref,
               m0_ref, m1_ref, m2_ref, mb0_ref, mb1_ref, mb2_ref,
               g1_ref, b1_ref, out_ref):
    new = jnp.dot(acc_ref[...], w2_ref[...], preferred_element_type=_f32)
    new = new / (den_ref[...] + 1e-9)
    t = _ln(old_ref[...] + new, g0_ref[...], b0_ref[...])
    h = jax.nn.relu(jnp.dot(t, m0_ref[...], preferred_element_type=_f32) + mb0_ref[...])
    h = jax.nn.relu(jnp.dot(h, m1_ref[...], preferred_element_type=_f32) + mb1_ref[...])
    h = jnp.dot(h, m2_ref[...], preferred_element_type=_f32) + mb2_ref[...]
    out_ref[...] = _ln(t + h, g1_ref[...], b1_ref[...])


def _mat_spec(i):
    return pl.BlockSpec((FM, FM), lambda i: (0, 0))


_W_SPECS = [pl.BlockSpec((FM, FM), lambda i: (0, 0))] * 4 + \
           [pl.BlockSpec((1, FM), lambda i: (0, 0))] * 2


def _post_specs(br, old_map, acc_map):
    full_m = lambda i: (0, 0)
    return [
        pl.BlockSpec((br, FM), old_map),
        pl.BlockSpec((br, FM), acc_map),
        pl.BlockSpec((br, 1), acc_map),
        pl.BlockSpec((FM, FM), full_m),
        pl.BlockSpec((1, FM), full_m),
        pl.BlockSpec((1, FM), full_m),
        pl.BlockSpec((FM, FM), full_m),
        pl.BlockSpec((FM, FM), full_m),
        pl.BlockSpec((FM, FM), full_m),
        pl.BlockSpec((1, FM), full_m),
        pl.BlockSpec((1, FM), full_m),
        pl.BlockSpec((1, FM), full_m),
        pl.BlockSpec((1, FM), full_m),
        pl.BlockSpec((1, FM), full_m),
    ]


def _post_c(old, acc, den, *weights):
    ident = lambda i: (i, 0)
    return pl.pallas_call(
        _post_body,
        grid=(25,),
        in_specs=_post_specs(2000, ident, ident),
        out_specs=pl.BlockSpec((2000, FM), ident),
        out_shape=jax.ShapeDtypeStruct((NC, FM), _f32),
    )(old, acc, den, *weights)


def _post_l(old, acc, den, *weights):
    ident = lambda i: (i, 0)
    flip = lambda i: (jnp.where(i < 25, i + 25, i - 25), 0)
    return pl.pallas_call(
        _post_body,
        grid=(50,),
        in_specs=_post_specs(1000, ident, flip),
        out_specs=pl.BlockSpec((1000, FM), ident),
        out_shape=jax.ShapeDtypeStruct((NL, FM), _f32),
    )(old, acc, den, *weights)


def _readout_body(lp_ref, ln_ref, w1_ref, b1_ref, w2_ref, b2_ref,
                  w3_ref, b3_ref, log_ref, sp_ref, sn_ref):
    v = jnp.concatenate([lp_ref[...], ln_ref[...]], axis=1)
    h = jax.nn.relu(jnp.dot(v, w1_ref[...], preferred_element_type=_f32) + b1_ref[...])
    h = jax.nn.relu(jnp.dot(h, w2_ref[...], preferred_element_type=_f32) + b2_ref[...])
    lg = jnp.dot(h, w3_ref[...], preferred_element_type=_f32) + b3_ref[...]
    log_ref[...] = lg
    sp_ref[...] = _softplus(lg)
    sn_ref[...] = _softplus(-lg)


def _readout(l_out, w1, b1, w2, b2, w3, b3):
    full_m = lambda i: (0, 0)
    return pl.pallas_call(
        _readout_body,
        grid=(25,),
        in_specs=[
            pl.BlockSpec((1000, FM), lambda i: (i, 0)),
            pl.BlockSpec((1000, FM), lambda i: (i + 25, 0)),
            pl.BlockSpec((2 * FM, 2 * FM), full_m),
            pl.BlockSpec((1, 2 * FM), full_m),
            pl.BlockSpec((2 * FM, 2 * FM), full_m),
            pl.BlockSpec((1, 2 * FM), full_m),
            pl.BlockSpec((2 * FM, 1), full_m),
            pl.BlockSpec((1, 1), full_m),
        ],
        out_specs=[
            pl.BlockSpec((1000, 1), lambda i: (i, 0)),
            pl.BlockSpec((1000, 1), lambda i: (i, 0)),
            pl.BlockSpec((1000, 1), lambda i: (i, 0)),
        ],
        out_shape=[
            jax.ShapeDtypeStruct((NV, 1), _f32),
            jax.ShapeDtypeStruct((NV, 1), _f32),
            jax.ShapeDtypeStruct((NV, 1), _f32),
        ],
    )(l_out, l_out, w1, b1, w2, b2, w3, b3)


def _loss_body(s_ref, out_ref):
    s = s_ref[...]
    cv = jnp.exp(-s)
    lg = jnp.log(1.0 - cv + EPS_LOSS)
    out_ref[...] = jnp.sum(lg * lg).reshape(1, 1)


def _loss(s_2d):
    return pl.pallas_call(
        _loss_body,
        out_shape=jax.ShapeDtypeStruct((1, 1), _f32),
    )(s_2d)


def kernel(edge_index, n_lits, n_clauses, L_init, C_init, lit_mlp_W, lit_mlp_b,
           cl_mlp_W, cl_mlp_b, att_l_W, att_c_W, ln_gamma, ln_beta,
           out_W1, out_b1, out_W2, out_b2, out_W3, out_b3):
    lit = edge_index[0]
    cl = edge_index[1]
    npad = NE_PAD - NE
    big = jnp.full((npad,), 2 ** 30, _i32)
    zpad = jnp.zeros((npad,), _i32)
    lit_r = jnp.concatenate([lit, big])
    lit_s = jnp.concatenate([lit, zpad])
    cl_r = jnp.concatenate([cl, big])
    cl_s = jnp.concatenate([cl, zpad])

    denom = jnp.sqrt(jnp.float32(FM))
    l_out = jnp.tile((L_init / denom).astype(_f32), (NL, 1))
    c_out = jnp.tile((C_init / denom).astype(_f32), (NC, 1))

    g = [ln_gamma[i].reshape(1, FM) for i in range(4)]
    b = [ln_beta[i].reshape(1, FM) for i in range(4)]
    cw = [cl_mlp_W[i] for i in range(3)]
    cb = [cl_mlp_b[i].reshape(1, FM) for i in range(3)]
    lw = [lit_mlp_W[i] for i in range(3)]
    lb = [lit_mlp_b[i].reshape(1, FM) for i in range(3)]
    b1r = out_b1.reshape(1, 2 * FM)
    b2r = out_b2.reshape(1, 2 * FM)
    b3r = out_b3.reshape(1, 1)

    losses = []
    logits = None
    for _ in range(2):
        qc = _qprime(c_out, att_c_W[0], att_c_W[1])
        accc, denc = _sc_attention(cl_r, lit_s, qc, l_out)
        c_out = _post_c(c_out, accc, denc.reshape(OUT_ROWS, 1), att_c_W[2],
                        g[0], b[0], cw[0], cw[1], cw[2], cb[0], cb[1], cb[2],
                        g[3], b[3])
        ql = _qprime(l_out, att_l_W[0], att_l_W[1])
        accl, denl = _sc_attention(lit_r, cl_s, ql, c_out)
        l_out = _post_l(l_out, accl, denl.reshape(OUT_ROWS, 1), att_l_W[2],
                        g[1], b[1], lw[0], lw[1], lw[2], lb[0], lb[1], lb[2],
                        g[2], b[2])
        logits, sp_pos, sp_neg = _readout(l_out, out_W1, b1r, out_W2, b2r,
                                          out_W3, b3r)
        lit_sp = jnp.concatenate([sp_pos, sp_neg], axis=0).reshape(NL)
        s_c = _sc_segsum(cl_r, lit_s, lit_sp)
        losses.append(_loss(s_c[:NC].reshape(400, 125))[0, 0])
    return logits, jnp.mean(jnp.stack(losses))
